# baseline (speedup 1.0000x reference)
import functools
import math

import jax
import jax.numpy as jnp
from jax import lax
from jax.experimental import pallas as pl
from jax.experimental.pallas import tpu as pltpu
from jax.experimental.pallas import tpu_sc as plsc

D_MODEL = 64
D_PAD = 128
LANES = 16
BL = 128
LU = 8


@functools.partial(jax.jit, static_argnames=("b", "l"))
def _sc_embed(xt, tpad, b, l):
    info = plsc.get_sparse_core_info()
    nw = info.num_cores * info.num_subcores
    blocks_per_l = b // BL
    n_units = (l // LU) * blocks_per_l
    units_per_w = n_units // nw
    n_subs = units_per_w * LU
    scale = math.sqrt(float(D_MODEL))
    mesh = plsc.VectorSubcoreMesh(core_axis_name="c", subcore_axis_name="s")

    @functools.partial(
        pl.kernel,
        mesh=mesh,
        out_type=jax.ShapeDtypeStruct((l, D_MODEL, b), jnp.float32),
        scratch_types=[
            pltpu.VMEM((2, LU, BL), jnp.int32),
            pltpu.VMEM((2, BL, D_PAD), jnp.float32),
            pltpu.VMEM((2, D_MODEL, BL), jnp.float32),
            pltpu.SemaphoreType.DMA,
            pltpu.SemaphoreType.DMA,
        ],
        compiler_params=pltpu.CompilerParams(
            use_tc_tiling_on_sc=True, needs_layout_passes=False),
    )
    def k(xt_hbm, tab_hbm, out_hbm, idx_v, rows_v, trans_v, sem_g, sem_w):
        wid = lax.axis_index("s") * info.num_cores + lax.axis_index("c")
        unit0 = wid * units_per_w
        lane = lax.iota(jnp.int32, LANES)
        b_base = [lane + j * LANES for j in range(BL // LANES)]

        def stage_idx(u):
            unit = unit0 + u
            l0 = (unit // blocks_per_l) * LU
            b0 = (unit % blocks_per_l) * BL
            pltpu.sync_copy(
                xt_hbm.at[pl.ds(l0, LU), pl.ds(b0, BL)], idx_v.at[u % 2])

        def gather_desc(s):
            u = s // LU
            return pltpu.make_async_copy(
                tab_hbm.at[idx_v.at[u % 2, s % LU]],
                rows_v.at[s % 2], sem_g)

        def write_desc(s):
            unit = unit0 + s // LU
            l_g = (unit // blocks_per_l) * LU + s % LU
            b0 = (unit % blocks_per_l) * BL
            return pltpu.make_async_copy(
                trans_v.at[s % 2],
                out_hbm.at[l_g, :, pl.ds(b0, BL)], sem_w)

        stage_idx(0)
        gather_desc(0).start()

        def sub_body(s, _):
            s1 = s + 1

            @pl.when(s1 < n_subs)
            def _fire_next():
                @pl.when(s1 % LU == 0)
                def _stage():
                    stage_idx(s1 // LU)
                gather_desc(s1).start()

            @pl.when(s >= 2)
            def _drain_write():
                write_desc(s - 2).wait()

            gather_desc(s).wait()
            buf = s % 2
            rows = rows_v.at[buf]

            @plsc.parallel_loop(0, D_MODEL, unroll=4)
            def d_body(d):
                col = jnp.full((LANES,), d, jnp.int32)
                for j in range(BL // LANES):
                    v = plsc.load_gather(rows, [b_base[j], col])
                    trans_v[buf, d, pl.ds(j * LANES, LANES)] = v
            write_desc(s).start()
            return 0

        lax.fori_loop(0, n_subs, sub_body, 0)
        write_desc(n_subs - 2).wait()
        write_desc(n_subs - 1).wait()

    return k(xt, tpad)


def kernel(x, table):
    b, l = x.shape
    tpad = jnp.pad(table * math.sqrt(float(D_MODEL)),
                   ((0, 0), (0, D_PAD - D_MODEL)))
    out_t = _sc_embed(x.T, tpad, b, l)
    return out_t.transpose(2, 0, 1)


# speedup vs baseline: 1.2627x; 1.2627x over previous
import functools
import math

import jax
import jax.numpy as jnp
from jax import lax
from jax.experimental import pallas as pl
from jax.experimental.pallas import tpu as pltpu
from jax.experimental.pallas import tpu_sc as plsc

D_MODEL = 64
D_PAD = 128
LANES = 16
BL = 128
LU = 8


@functools.partial(jax.jit, static_argnames=("b", "l"))
def _sc_embed(xt, tpad, b, l):
    info = plsc.get_sparse_core_info()
    nw = info.num_cores * info.num_subcores
    blocks_per_l = b // BL
    n_units = (l // LU) * blocks_per_l
    units_per_w = n_units // nw
    n_subs = units_per_w * LU
    scale = math.sqrt(float(D_MODEL))
    mesh = plsc.VectorSubcoreMesh(core_axis_name="c", subcore_axis_name="s")

    @functools.partial(
        pl.kernel,
        mesh=mesh,
        out_type=jax.ShapeDtypeStruct((l, D_MODEL, b), jnp.float32),
        scratch_types=[
            pltpu.VMEM((2, LU, BL), jnp.int32),
            pltpu.VMEM((2, BL, D_PAD), jnp.float32),
            pltpu.VMEM((2, D_MODEL, BL), jnp.float32),
            pltpu.SemaphoreType.DMA,
            pltpu.SemaphoreType.DMA,
        ],
        compiler_params=pltpu.CompilerParams(
            use_tc_tiling_on_sc=True, needs_layout_passes=False),
    )
    def k(xt_hbm, tab_hbm, out_hbm, idx_v, rows_v, trans_v, sem_g, sem_w):
        wid = lax.axis_index("s") * info.num_cores + lax.axis_index("c")
        unit0 = wid * units_per_w
        lane = lax.iota(jnp.int32, LANES)
        b_base = [lane + j * LANES for j in range(BL // LANES)]

        def stage_idx(u):
            unit = unit0 + u
            l0 = (unit // blocks_per_l) * LU
            b0 = (unit % blocks_per_l) * BL
            pltpu.sync_copy(
                xt_hbm.at[pl.ds(l0, LU), pl.ds(b0, BL)], idx_v.at[u % 2])

        def gather_desc(s):
            u = s // LU
            return pltpu.make_async_copy(
                tab_hbm.at[idx_v.at[u % 2, s % LU]],
                rows_v.at[s % 2], sem_g)

        def write_desc(s):
            unit = unit0 + s // LU
            l_g = (unit // blocks_per_l) * LU + s % LU
            b0 = (unit % blocks_per_l) * BL
            return pltpu.make_async_copy(
                trans_v.at[s % 2],
                out_hbm.at[l_g, :, pl.ds(b0, BL)], sem_w)

        stage_idx(0)
        gather_desc(0).start()

        def sub_body(s, _):
            s1 = s + 1

            @pl.when(s1 < n_subs)
            def _fire_next():
                @pl.when(s1 % LU == 0)
                def _stage():
                    stage_idx(s1 // LU)
                gather_desc(s1).start()

            @pl.when(s >= 2)
            def _drain_write():
                write_desc(s - 2).wait()

            gather_desc(s).wait()
            buf = s % 2
            rows = rows_v.at[buf]

            @plsc.parallel_loop(0, D_MODEL, unroll=4)
            def d_body(d):
                col = jnp.full((LANES,), d, jnp.int32)
                for j in range(BL // LANES):
                    v = plsc.load_gather(rows, [b_base[j], col])
                    trans_v[buf, d, pl.ds(j * LANES, LANES)] = v
            write_desc(s).start()
            return 0

        lax.fori_loop(0, n_subs, sub_body, 0)
        write_desc(n_subs - 2).wait()
        write_desc(n_subs - 1).wait()

    return k(xt, tpad)


def kernel(x, table):
    b, l = x.shape
    tpad = jnp.pad(table, ((0, 0), (0, D_PAD - D_MODEL))) * math.sqrt(
        float(D_MODEL))
    out_t = _sc_embed(x.T, tpad, b, l)
    return out_t.transpose(2, 0, 1)
